# baseline (speedup 1.0000x reference)
import functools

import jax
import jax.numpy as jnp
from jax import lax
from jax.experimental import pallas as pl
from jax.experimental.pallas import tpu as pltpu
from jax.experimental.pallas import tpu_sc as plsc

V = 1_000_000
D = 32
G = V // 4
RBG = 8_192
NSTEPS = -(-G // RBG)
TAIL = G - (NSTEPS - 1) * RBG

B = 204_800
CHUNK = 128
NW = 32
CPW = B // (NW * CHUNK)
GB = B // 4
EPS = 1e-12


RB2 = 16_384
NS2 = -(-V // RB2)
TAIL2 = V - (NS2 - 1) * RB2


def _sigma_body(u_hbm, ut_ref, w_hbm, sig_ref, acc_m, acc_t, uscr, wscr,
                usem, wsem):
    i = pl.program_id(0)

    def _wdma(j, buf, nrows):
        return pltpu.make_async_copy(
            w_hbm.at[pl.ds(j * RB2, nrows), :],
            wscr.at[buf, pl.ds(0, nrows), :], wsem)

    @pl.when(i == 0)
    def _init():
        acc_m[...] = jnp.zeros_like(acc_m)
        acc_t[...] = jnp.zeros_like(acc_t)
        _wdma(0, 0, RB2).start()

    @pl.when(i < NS2 - 1)
    def _ldfull():
        cp = pltpu.make_async_copy(
            u_hbm.at[:, pl.ds(i * RB2, RB2)], uscr, usem)
        cp.start()
        cp.wait()

    @pl.when(i < NS2 - 1)
    def _wwaitf():
        _wdma(i, i % 2, RB2).wait()

    @pl.when(i == NS2 - 1)
    def _wwaitt():
        _wdma(i, i % 2, TAIL2).wait()

    @pl.when(i < NS2 - 2)
    def _wnextf():
        _wdma(i + 1, (i + 1) % 2, RB2).start()

    @pl.when(i == NS2 - 2)
    def _wnextt():
        _wdma(i + 1, (i + 1) % 2, TAIL2).start()

    def _acc(wb, ub):
        acc_m[...] += lax.dot_general(wb, wb, (((0,), (0,)), ((), ())),
                                      preferred_element_type=jnp.float32)
        acc_t[...] += lax.dot_general(ub, wb, (((1,), (0,)), ((), ())),
                                      preferred_element_type=jnp.float32)

    @pl.when(i < NS2 - 1)
    def _full():
        _acc(wscr[i % 2], uscr[...])

    @pl.when(i == NS2 - 1)
    def _tail():
        rows = lax.broadcasted_iota(jnp.int32, (RB2, D), 0)
        _acc(jnp.where(rows < TAIL2, wscr[i % 2], 0.0), ut_ref[...])

    @pl.when(i == NS2 - 1)
    def _fin():
        m = acc_m[...]
        t = acc_t[...]
        nt = jnp.sqrt(jnp.sum(t * t))
        v = t / (nt + EPS)
        mv = lax.dot_general(v, m, (((1,), (0,)), ((), ())),
                             preferred_element_type=jnp.float32,
                             precision=lax.Precision.HIGHEST)
        q = jnp.sum(mv * v)
        sig_ref[0, 0] = q / (jnp.sqrt(q) + EPS)


def _sigma(weight, u, u_tail):
    return pl.pallas_call(
        _sigma_body,
        grid=(NS2,),
        in_specs=[
            pl.BlockSpec(memory_space=pltpu.MemorySpace.HBM),
            pl.BlockSpec((1, RB2), lambda i: (0, 0)),
            pl.BlockSpec(memory_space=pltpu.MemorySpace.HBM),
        ],
        out_specs=pl.BlockSpec(memory_space=pltpu.MemorySpace.SMEM),
        out_shape=jax.ShapeDtypeStruct((1, 1), jnp.float32),
        scratch_shapes=[
            pltpu.VMEM((D, D), jnp.float32),
            pltpu.VMEM((1, D), jnp.float32),
            pltpu.VMEM((1, RB2), jnp.float32),
            pltpu.VMEM((2, RB2, D), jnp.float32),
            pltpu.SemaphoreType.DMA,
            pltpu.SemaphoreType.DMA,
        ],
    )(u, u_tail, weight)


def _gather_body(idx_hbm, table_hbm, out_hbm, idx_v, sidx_v, sbuf, obuf, gsem):
    cid = lax.axis_index("c")
    sid = lax.axis_index("s")
    wid = sid * 2 + cid
    ipw = CPW * CHUNK
    row0 = wid * CPW

    pltpu.sync_copy(idx_hbm.at[pl.ds(wid * ipw, ipw)], idx_v)

    def mk_sidx(j, carry):
        for k in range(8):
            s = pl.ds(j * CHUNK + k * 16, 16)
            sidx_v[s] = lax.shift_right_logical(idx_v[s], 2)
        return carry

    lax.fori_loop(0, CPW, mk_sidx, 0)

    def chunk(j, carry):
        pltpu.async_copy(
            table_hbm.at[sidx_v.at[pl.ds(j * CHUNK, CHUNK)]], sbuf, gsem
        ).wait()
        for g in range(CHUNK // 16):
            offs = (idx_v[pl.ds(j * CHUNK + g * 16, 16)] & 3) * D
            for k in range(16):
                r = g * 16 + k
                off = offs[k]
                obuf[r // 4, pl.ds((r % 4) * D, 16)] = sbuf[r, pl.ds(off, 16)]
                obuf[r // 4, pl.ds((r % 4) * D + 16, 16)] = sbuf[r, pl.ds(off + 16, 16)]
        pltpu.sync_copy(obuf, out_hbm.at[pl.ds((row0 + j) * 32, 32)])
        return carry

    lax.fori_loop(0, CPW, chunk, 0)


def _gather(idx1d, table128):
    mesh = plsc.VectorSubcoreMesh(core_axis_name="c", subcore_axis_name="s",
                                  num_cores=2, num_subcores=16)
    return pl.kernel(
        _gather_body,
        out_type=jax.ShapeDtypeStruct((GB, 4 * D), jnp.float32),
        mesh=mesh,
        scratch_types=[
            pltpu.VMEM((CPW * CHUNK,), jnp.int32),
            pltpu.VMEM((CPW * CHUNK,), jnp.int32),
            pltpu.VMEM((CHUNK, 4 * D), jnp.float32),
            pltpu.VMEM((CHUNK // 4, 4 * D), jnp.float32),
            pltpu.SemaphoreType.DMA,
        ],
    )(idx1d, table128)


def _scale_body(sig_ref, x_ref, o_ref):
    o_ref[...] = x_ref[...] * (1.0 / sig_ref[0, 0])


def _scale(sigma, raw):
    blk = 2048
    return pl.pallas_call(
        _scale_body,
        grid=(GB // blk,),
        in_specs=[
            pl.BlockSpec(memory_space=pltpu.MemorySpace.SMEM),
            pl.BlockSpec((blk, 4 * D), lambda i: (i, 0)),
        ],
        out_specs=pl.BlockSpec((blk, 4 * D), lambda i: (i, 0)),
        out_shape=jax.ShapeDtypeStruct((GB, 4 * D), jnp.float32),
    )(sigma, raw)


def kernel(input, weight, u):
    idx1d = input.reshape(B).astype(jnp.int32)
    table128 = weight.reshape(G, 4 * D)
    u_tail = jnp.pad(lax.slice(u, (0, (NS2 - 1) * RB2), (1, V)),
                     ((0, 0), (0, RB2 - TAIL2)))
    sigma = _sigma(weight, u, u_tail)
    raw = _gather(idx1d, table128)
    out = _scale(sigma, raw)
    return out.reshape(input.shape + (D,))


# speedup vs baseline: 1.2976x; 1.2976x over previous
import functools

import jax
import jax.numpy as jnp
from jax import lax
from jax.experimental import pallas as pl
from jax.experimental.pallas import tpu as pltpu
from jax.experimental.pallas import tpu_sc as plsc

V = 1_000_000
D = 32
G = V // 4
RBG = 8_192
NSTEPS = -(-G // RBG)
TAIL = G - (NSTEPS - 1) * RBG

B = 204_800
CHUNK = 128
NW = 32
CPW = B // (NW * CHUNK)
GB = B // 4
EPS = 1e-12


def _sigma_body(u_ref, w_ref, sig_ref, acc_p, acc_z, umask):
    i = pl.program_id(0)
    SB = RBG // 32

    @pl.when(i == 0)
    def _init():
        acc_p[...] = jnp.zeros_like(acc_p)
        acc_z[...] = jnp.zeros_like(acc_z)
        lane = lax.broadcasted_iota(jnp.int32, (RBG, 4 * D), 1)
        row = lax.broadcasted_iota(jnp.int32, (RBG, 4 * D), 0)
        umask[...] = jnp.where((lane >> 2) == (row & 31), 1.0, 0.0)

    def _acc(wb, ub):
        acc_p[...] += lax.dot_general(wb, wb, (((0,), (0,)), ((), ())),
                                      preferred_element_type=jnp.float32)
        u128 = ub.reshape(SB, 4 * D)
        urep = jnp.broadcast_to(u128[:, None, :], (SB, 32, 4 * D))
        c = urep.reshape(RBG, 4 * D) * umask[...]
        acc_z[...] += lax.dot_general(c, wb, (((0,), (0,)), ((), ())),
                                      preferred_element_type=jnp.float32)

    @pl.when(i < NSTEPS - 1)
    def _full():
        _acc(w_ref[...], u_ref[...])

    @pl.when(i == NSTEPS - 1)
    def _tail():
        rows = lax.broadcasted_iota(jnp.int32, (RBG, 4 * D), 0)
        lanes = lax.broadcasted_iota(jnp.int32, (1, 4 * RBG), 1)
        _acc(jnp.where(rows < TAIL, w_ref[...], 0.0),
             jnp.where(lanes < 4 * TAIL, u_ref[...], 0.0))

    @pl.when(i == NSTEPS - 1)
    def _fin():
        p = acc_p[...]
        m = (p[0:D, 0:D] + p[D:2 * D, D:2 * D]
             + p[2 * D:3 * D, 2 * D:3 * D] + p[3 * D:4 * D, 3 * D:4 * D])
        z = acc_z[...]
        t = jnp.zeros((1, D), jnp.float32)
        for k in range(4 * D):
            a = k & 3
            t = t + z[k:k + 1, a * D:(a + 1) * D]
        nt = jnp.sqrt(jnp.sum(t * t))
        v = t / (nt + EPS)
        mv = lax.dot_general(v, m, (((1,), (0,)), ((), ())),
                             preferred_element_type=jnp.float32,
                             precision=lax.Precision.HIGHEST)
        q = jnp.sum(mv * v)
        sig_ref[0, 0] = q / (jnp.sqrt(q) + EPS)


def _sigma(table128, u):
    return pl.pallas_call(
        _sigma_body,
        grid=(NSTEPS,),
        in_specs=[
            pl.BlockSpec((1, 4 * RBG), lambda i: (0, i)),
            pl.BlockSpec((RBG, 4 * D), lambda i: (i, 0)),
        ],
        out_specs=pl.BlockSpec(memory_space=pltpu.MemorySpace.SMEM),
        out_shape=jax.ShapeDtypeStruct((1, 1), jnp.float32),
        scratch_shapes=[
            pltpu.VMEM((4 * D, 4 * D), jnp.float32),
            pltpu.VMEM((4 * D, 4 * D), jnp.float32),
            pltpu.VMEM((RBG, 4 * D), jnp.float32),
        ],
    )(u, table128)


def _gather_body(idx_hbm, table_hbm, out_hbm, idx_v, sidx_v, sbuf, obuf, gsem):
    cid = lax.axis_index("c")
    sid = lax.axis_index("s")
    wid = sid * 2 + cid
    ipw = CPW * CHUNK
    row0 = wid * CPW

    pltpu.sync_copy(idx_hbm.at[pl.ds(wid * ipw, ipw)], idx_v)

    def mk_sidx(j, carry):
        for k in range(8):
            s = pl.ds(j * CHUNK + k * 16, 16)
            sidx_v[s] = lax.shift_right_logical(idx_v[s], 2)
        return carry

    lax.fori_loop(0, CPW, mk_sidx, 0)

    def chunk(j, carry):
        pltpu.async_copy(
            table_hbm.at[sidx_v.at[pl.ds(j * CHUNK, CHUNK)]], sbuf, gsem
        ).wait()
        for g in range(CHUNK // 16):
            offs = (idx_v[pl.ds(j * CHUNK + g * 16, 16)] & 3) * D
            for k in range(16):
                r = g * 16 + k
                off = offs[k]
                obuf[r // 4, pl.ds((r % 4) * D, 16)] = sbuf[r, pl.ds(off, 16)]
                obuf[r // 4, pl.ds((r % 4) * D + 16, 16)] = sbuf[r, pl.ds(off + 16, 16)]
        pltpu.sync_copy(obuf, out_hbm.at[pl.ds((row0 + j) * 32, 32)])
        return carry

    lax.fori_loop(0, CPW, chunk, 0)


def _gather(idx1d, table128):
    mesh = plsc.VectorSubcoreMesh(core_axis_name="c", subcore_axis_name="s",
                                  num_cores=2, num_subcores=16)
    return pl.kernel(
        _gather_body,
        out_type=jax.ShapeDtypeStruct((GB, 4 * D), jnp.float32),
        mesh=mesh,
        scratch_types=[
            pltpu.VMEM((CPW * CHUNK,), jnp.int32),
            pltpu.VMEM((CPW * CHUNK,), jnp.int32),
            pltpu.VMEM((CHUNK, 4 * D), jnp.float32),
            pltpu.VMEM((CHUNK // 4, 4 * D), jnp.float32),
            pltpu.SemaphoreType.DMA,
        ],
    )(idx1d, table128)


def _scale_body(sig_ref, x_ref, o_ref):
    o_ref[...] = x_ref[...] * (1.0 / sig_ref[0, 0])


def _scale(sigma, raw):
    blk = 2048
    return pl.pallas_call(
        _scale_body,
        grid=(GB // blk,),
        in_specs=[
            pl.BlockSpec(memory_space=pltpu.MemorySpace.SMEM),
            pl.BlockSpec((blk, 4 * D), lambda i: (i, 0)),
        ],
        out_specs=pl.BlockSpec((blk, 4 * D), lambda i: (i, 0)),
        out_shape=jax.ShapeDtypeStruct((GB, 4 * D), jnp.float32),
    )(sigma, raw)


def kernel(input, weight, u):
    idx1d = input.reshape(B).astype(jnp.int32)
    table128 = weight.reshape(G, 4 * D)
    sigma = _sigma(table128, u)
    raw = _gather(idx1d, table128)
    out = _scale(sigma, raw)
    return out.reshape(input.shape + (D,))
